# baseline (speedup 1.0000x reference)
import jax
import jax.numpy as jnp
from jax.experimental import pallas as pl
from jax.experimental.pallas import tpu as pltpu

_ROWS = 1536
_COLS = 512
_B_PER_BLOCK = 2


def _qsample_body(t_ref, sac_ref, somac_ref, x_ref, n_ref, o_ref):
    bb = pl.program_id(0)
    for k in range(_B_PER_BLOCK):
        tt = t_ref[bb * _B_PER_BLOCK + k]
        a = sac_ref[tt]
        s = somac_ref[tt]
        o_ref[k] = a * x_ref[k] + s * n_ref[k]


def kernel(x_start, t, noise, sqrt_alphas_cumprod, sqrt_one_minus_alphas_cumprod):
    B, C, H, W = x_start.shape
    xr = x_start.reshape(B, _ROWS, _COLS)
    nr = noise.reshape(B, _ROWS, _COLS)
    t32 = t.astype(jnp.int32)

    grid = (B // _B_PER_BLOCK,)
    spec = pl.BlockSpec((_B_PER_BLOCK, _ROWS, _COLS), lambda b, *_: (b, 0, 0))
    grid_spec = pltpu.PrefetchScalarGridSpec(
        num_scalar_prefetch=3,
        grid=grid,
        in_specs=[spec, spec],
        out_specs=spec,
    )
    out = pl.pallas_call(
        _qsample_body,
        grid_spec=grid_spec,
        out_shape=jax.ShapeDtypeStruct((B, _ROWS, _COLS), jnp.float32),
    )(t32, sqrt_alphas_cumprod, sqrt_one_minus_alphas_cumprod, xr, nr)
    return out.reshape(B, C, H, W)
